# baseline (speedup 1.0000x reference)
import functools

import jax
import jax.numpy as jnp
from jax import lax
from jax.experimental import pallas as pl
from jax.experimental.pallas import tpu as pltpu
from jax.experimental.pallas import tpu_sc as plsc

_GAMMA = 12.0
_BATCH = 1024
_NEG = 256
_D = 64
_W = 128
_NE = 1000000
_THR = 503808
_TBLK = 4096
_TGRID = _THR // _TBLK
_NC = 2
_NS = 16
_NW = _NC * _NS
_BPW = _BATCH // _NW
_CHUNK = 128
_NCHUNK = _BPW * _NEG // _CHUNK
_NBUF = 4
_L = 16


def _tp_body(x1_ref, x2_ref, y_ref):
    for s in range(_TBLK // 256):
        sl = pl.ds(s * 256, 256)
        y_ref[sl, :] = jnp.concatenate(
            [x1_ref[:, sl].T, x2_ref[:, sl].T], axis=1)


@jax.jit
def _pair_table(ent_t):
    return pl.pallas_call(
        _tp_body,
        grid=(_TGRID,),
        in_specs=[
            pl.BlockSpec((_D, _TBLK), lambda i: (0, i)),
            pl.BlockSpec((_D, _TBLK),
                         lambda i: (0, jnp.minimum(_TGRID + i,
                                                   (_NE + _TBLK - 1)
                                                   // _TBLK - 1))),
        ],
        out_specs=pl.BlockSpec((_TBLK, _W), lambda i: (i, 0)),
        out_shape=jax.ShapeDtypeStruct((_THR, _W), jnp.float32),
    )(ent_t, ent_t)


def _sc_body(hp_hbm, tail_hbm, ptab_hbm, htab_hbm, rtab_hbm, out_hbm,
             hp_v, hidx_v, ridx_v, hbuf, rbuf, tidx_v, loff_v,
             tb0, tb1, tb2, tb3, tscr, scores_v,
             sem0, sem1, sem2, sem3, semh):
    bufs = (tb0, tb1, tb2, tb3)
    sems = (sem0, sem1, sem2, sem3)
    wid = lax.axis_index("s") * _NC + lax.axis_index("c")
    b0 = wid * _BPW
    iota = lax.broadcasted_iota(jnp.int32, (_L,), 0)

    pltpu.sync_copy(hp_hbm.at[pl.ds(b0 * 3, _BPW * 3)], hp_v)
    pltpu.sync_copy(tail_hbm.at[pl.ds(wid * _NCHUNK, _NCHUNK)], tidx_v)

    for h in range(_BPW // _L):
        pos = (iota + h * _L) * 3
        hidx_v[pl.ds(h * _L, _L)] = plsc.load_gather(hp_v, [pos])
        ridx_v[pl.ds(h * _L, _L)] = plsc.load_gather(hp_v, [pos + 1])

    def _split(j, _):
        for h in range(_CHUNK // _L):
            sl = pl.ds(h * _L, _L)
            e = tidx_v[j, sl]
            ge = e >= _THR
            tidx_v[j, sl] = jnp.where(ge, e - _THR, e)
            loff_v[j, sl] = jnp.where(ge, _D, 0)
        return 0
    lax.fori_loop(0, _NCHUNK, _split, 0)

    pltpu.async_copy(htab_hbm.at[hidx_v], hbuf, semh).wait()
    pltpu.async_copy(rtab_hbm.at[ridx_v], rbuf, semh).wait()

    def _hr_add(i, _):
        for c in range(_D // _L):
            sl = pl.ds(c * _L, _L)
            hbuf[i, sl] = hbuf[i, sl] + rbuf[i, sl]
        return 0
    lax.fori_loop(0, _BPW, _hr_add, 0)

    def _fire(j, buf, sem):
        pltpu.async_copy(ptab_hbm.at[tidx_v.at[j]], buf, sem)

    def _wait(buf, sem):
        pltpu.make_async_copy(ptab_hbm.at[tidx_v.at[0]], buf, sem).wait()

    for p in range(_NBUF):
        _fire(p, bufs[p], sems[p])

    def _absdiff(a, b):
        d = plsc.bitcast(a - b, jnp.int32) & jnp.int32(0x7FFFFFFF)
        return plsc.bitcast(d, jnp.float32)

    def _compute_chunk(jj, half, buf):
        hrow = [hbuf[jj, pl.ds(c * _L, _L)] for c in range(_D // _L)]
        j = 2 * jj + half

        def _group(g):
            offv = loff_v[j, pl.ds(g * _L, _L)]
            for r in range(_L):
                rowv = jnp.full((_L,), g * _L + r, jnp.int32)
                idx0 = offv[r] + iota
                tv = [plsc.load_gather(buf, [rowv, idx0 + c * _L])
                      for c in range(_D // _L)]
                ds = [_absdiff(hrow[c], tv[c]) for c in range(_D // _L)]
                tscr[r, pl.ds(0, _L)] = (ds[0] + ds[1]) + (ds[2] + ds[3])
            cols = [plsc.load_gather(tscr, [iota, jnp.full((_L,), dd,
                                                           jnp.int32)])
                    for dd in range(_L)]
            for w in (8, 4, 2, 1):
                cols = [cols[k] + cols[k + w] for k in range(w)]
            scores_v[jj, pl.ds(half * _CHUNK + g * _L, _L)] = (
                jnp.float32(_GAMMA) - cols[0])

        def _group2(h, _):
            _group(2 * h)
            _group(2 * h + 1)
            return 0
        lax.fori_loop(0, _CHUNK // (2 * _L), _group2, 0)

    def _main(jj, _):
        for p in range(_NBUF):
            j4 = _NBUF * jj + p
            buf, sem = bufs[p], sems[p]
            _wait(buf, sem)
            _compute_chunk(j4 // 2, j4 % 2, buf)

            @pl.when(jj < _NCHUNK // _NBUF - 1)
            def _():
                _fire(j4 + _NBUF, buf, sem)
        return 0
    lax.fori_loop(0, _NCHUNK // _NBUF, _main, 0)

    pltpu.sync_copy(scores_v, out_hbm.at[pl.ds(b0, _BPW)])


@jax.jit
def _sc_scores(hp_flat, tail_r, ptab, htab, rtab):
    mesh = plsc.VectorSubcoreMesh(core_axis_name="c", subcore_axis_name="s",
                                  num_cores=_NC, num_subcores=_NS)
    return pl.kernel(
        _sc_body,
        out_type=jax.ShapeDtypeStruct((_BATCH, _NEG), jnp.float32),
        mesh=mesh,
        compiler_params=pltpu.CompilerParams(needs_layout_passes=False,
                                             use_tc_tiling_on_sc=True),
        scratch_types=[
            pltpu.VMEM((_BPW * 3,), jnp.int32),
            pltpu.VMEM((_BPW,), jnp.int32),
            pltpu.VMEM((_BPW,), jnp.int32),
            pltpu.VMEM((_BPW, _W), jnp.float32),
            pltpu.VMEM((_BPW, _W), jnp.float32),
            pltpu.VMEM((_NCHUNK, _CHUNK), jnp.int32),
            pltpu.VMEM((_NCHUNK, _CHUNK), jnp.int32),
            pltpu.VMEM((_CHUNK, _W), jnp.float32),
            pltpu.VMEM((_CHUNK, _W), jnp.float32),
            pltpu.VMEM((_CHUNK, _W), jnp.float32),
            pltpu.VMEM((_CHUNK, _W), jnp.float32),
            pltpu.VMEM((_L, 17), jnp.float32),
            pltpu.VMEM((_BPW, _NEG), jnp.float32),
            pltpu.SemaphoreType.DMA,
            pltpu.SemaphoreType.DMA,
            pltpu.SemaphoreType.DMA,
            pltpu.SemaphoreType.DMA,
            pltpu.SemaphoreType.DMA,
        ],
    )(hp_flat, tail_r, ptab, htab, rtab)


def kernel(head_part, tail_part, edge_reltype, entity_embedding,
           relation_embedding):
    del edge_reltype
    hp_flat = head_part.reshape(-1)
    tail_r = tail_part.reshape(_NW * _NCHUNK, _CHUNK)
    ptab = _pair_table(entity_embedding.T)
    htab = jnp.pad(entity_embedding[:1000], ((0, 0), (0, _W - _D)))
    rtab = jnp.pad(relation_embedding, ((0, 0), (0, _W - _D)))
    return _sc_scores(hp_flat, tail_r, ptab, htab, rtab)


# speedup vs baseline: 1.0244x; 1.0244x over previous
import functools

import jax
import jax.numpy as jnp
from jax import lax
from jax.experimental import pallas as pl
from jax.experimental.pallas import tpu as pltpu
from jax.experimental.pallas import tpu_sc as plsc

_GAMMA = 12.0
_BATCH = 1024
_NEG = 256
_D = 64
_W = 128
_NE = 1000000
_THR = 503808
_TBLK = 4096
_TGRID = _THR // _TBLK
_NC = 2
_NS = 16
_NW = _NC * _NS
_BPW = _BATCH // _NW
_CHUNK = 128
_NCHUNK = _BPW * _NEG // _CHUNK
_NBUF = 4
_L = 16


def _tp_body(x1_ref, x2_ref, y_ref):
    for s in range(_TBLK // 256):
        sl = pl.ds(s * 256, 256)
        y_ref[sl, :] = jnp.concatenate(
            [x1_ref[:, sl].T, x2_ref[:, sl].T], axis=1)


@jax.jit
def _pair_table(ent_t):
    return pl.pallas_call(
        _tp_body,
        grid=(_TGRID,),
        in_specs=[
            pl.BlockSpec((_D, _TBLK), lambda i: (0, i)),
            pl.BlockSpec((_D, _TBLK),
                         lambda i: (0, jnp.minimum(_TGRID + i,
                                                   (_NE + _TBLK - 1)
                                                   // _TBLK - 1))),
        ],
        out_specs=pl.BlockSpec((_TBLK, _W), lambda i: (i, 0)),
        out_shape=jax.ShapeDtypeStruct((_THR, _W), jnp.float32),
    )(ent_t, ent_t)


def _sc_body(hp_hbm, tail_hbm, ptab_hbm, htab_hbm, rtab_hbm, out_hbm,
             hp_v, hidx_v, ridx_v, hbuf, rbuf, tidx_v, loff_v,
             tb0, tb1, tb2, tb3, tscr, scores_v,
             sem0, sem1, sem2, sem3, semh):
    bufs = (tb0, tb1, tb2, tb3)
    sems = (sem0, sem1, sem2, sem3)
    wid = lax.axis_index("s") * _NC + lax.axis_index("c")
    b0 = wid * _BPW
    iota = lax.broadcasted_iota(jnp.int32, (_L,), 0)

    pltpu.sync_copy(hp_hbm.at[pl.ds(b0 * 3, _BPW * 3)], hp_v)
    pltpu.sync_copy(tail_hbm.at[pl.ds(wid * _NCHUNK, _NCHUNK)], tidx_v)

    for h in range(_BPW // _L):
        pos = (iota + h * _L) * 3
        hidx_v[pl.ds(h * _L, _L)] = plsc.load_gather(hp_v, [pos])
        ridx_v[pl.ds(h * _L, _L)] = plsc.load_gather(hp_v, [pos + 1])

    def _split(j, _):
        for h in range(_CHUNK // _L):
            sl = pl.ds(h * _L, _L)
            e = tidx_v[j, sl]
            ge = e >= _THR
            tidx_v[j, sl] = jnp.where(ge, e - _THR, e)
            loff_v[j, sl] = jnp.where(ge, _D, 0)
        return 0
    lax.fori_loop(0, _NCHUNK, _split, 0)

    pltpu.async_copy(htab_hbm.at[hidx_v], hbuf, semh).wait()
    pltpu.async_copy(rtab_hbm.at[ridx_v], rbuf, semh).wait()

    def _hr_add(i, _):
        for c in range(_D // _L):
            sl = pl.ds(c * _L, _L)
            hbuf[i, sl] = hbuf[i, sl] + rbuf[i, sl]
        return 0
    lax.fori_loop(0, _BPW, _hr_add, 0)

    def _fire(j, buf, sem):
        pltpu.async_copy(ptab_hbm.at[tidx_v.at[j]], buf, sem)

    def _wait(buf, sem):
        pltpu.make_async_copy(ptab_hbm.at[tidx_v.at[0]], buf, sem).wait()

    for p in range(_NBUF):
        _fire(p, bufs[p], sems[p])

    def _absdiff(a, b):
        d = plsc.bitcast(a - b, jnp.int32) & jnp.int32(0x7FFFFFFF)
        return plsc.bitcast(d, jnp.float32)

    def _compute_chunk(jj, half, buf):
        hrow = [hbuf[jj, pl.ds(c * _L, _L)] for c in range(_D // _L)]
        j = 2 * jj + half

        def _group(g, _):
            offv = loff_v[j, pl.ds(g * _L, _L)]
            for r in range(_L):
                rowv = jnp.full((_L,), g * _L + r, jnp.int32)
                idx0 = offv[r] + iota
                tv = [plsc.load_gather(buf, [rowv, idx0 + c * _L])
                      for c in range(_D // _L)]
                ds = [_absdiff(hrow[c], tv[c]) for c in range(_D // _L)]
                tscr[r, pl.ds(0, _L)] = (ds[0] + ds[1]) + (ds[2] + ds[3])
            cols = [plsc.load_gather(tscr, [iota, jnp.full((_L,), dd,
                                                           jnp.int32)])
                    for dd in range(_L)]
            for w in (8, 4, 2, 1):
                cols = [cols[k] + cols[k + w] for k in range(w)]
            scores_v[jj, pl.ds(half * _CHUNK + g * _L, _L)] = (
                jnp.float32(_GAMMA) - cols[0])
            return 0
        lax.fori_loop(0, _CHUNK // _L, _group, 0)

    def _main(jj, _):
        for p in range(_NBUF):
            j4 = _NBUF * jj + p
            buf, sem = bufs[p], sems[p]
            _wait(buf, sem)
            _compute_chunk(j4 // 2, j4 % 2, buf)

            @pl.when(jj < _NCHUNK // _NBUF - 1)
            def _():
                _fire(j4 + _NBUF, buf, sem)
        return 0
    lax.fori_loop(0, _NCHUNK // _NBUF, _main, 0)

    pltpu.sync_copy(scores_v, out_hbm.at[pl.ds(b0, _BPW)])


@jax.jit
def _sc_scores(hp_flat, tail_r, ptab, htab, rtab):
    mesh = plsc.VectorSubcoreMesh(core_axis_name="c", subcore_axis_name="s",
                                  num_cores=_NC, num_subcores=_NS)
    return pl.kernel(
        _sc_body,
        out_type=jax.ShapeDtypeStruct((_BATCH, _NEG), jnp.float32),
        mesh=mesh,
        compiler_params=pltpu.CompilerParams(needs_layout_passes=False,
                                             use_tc_tiling_on_sc=True),
        scratch_types=[
            pltpu.VMEM((_BPW * 3,), jnp.int32),
            pltpu.VMEM((_BPW,), jnp.int32),
            pltpu.VMEM((_BPW,), jnp.int32),
            pltpu.VMEM((_BPW, _W), jnp.float32),
            pltpu.VMEM((_BPW, _W), jnp.float32),
            pltpu.VMEM((_NCHUNK, _CHUNK), jnp.int32),
            pltpu.VMEM((_NCHUNK, _CHUNK), jnp.int32),
            pltpu.VMEM((_CHUNK, _W), jnp.float32),
            pltpu.VMEM((_CHUNK, _W), jnp.float32),
            pltpu.VMEM((_CHUNK, _W), jnp.float32),
            pltpu.VMEM((_CHUNK, _W), jnp.float32),
            pltpu.VMEM((_L, 17), jnp.float32),
            pltpu.VMEM((_BPW, _NEG), jnp.float32),
            pltpu.SemaphoreType.DMA,
            pltpu.SemaphoreType.DMA,
            pltpu.SemaphoreType.DMA,
            pltpu.SemaphoreType.DMA,
            pltpu.SemaphoreType.DMA,
        ],
    )(hp_flat, tail_r, ptab, htab, rtab)


def kernel(head_part, tail_part, edge_reltype, entity_embedding,
           relation_embedding):
    del edge_reltype
    hp_flat = head_part.reshape(-1)
    tail_r = tail_part.reshape(_NW * _NCHUNK, _CHUNK)
    ptab = _pair_table(entity_embedding.T)
    htab = jnp.pad(entity_embedding[:1000], ((0, 0), (0, _W - _D)))
    rtab = jnp.pad(relation_embedding, ((0, 0), (0, _W - _D)))
    return _sc_scores(hp_flat, tail_r, ptab, htab, rtab)
